# baseline (speedup 1.0000x reference)
import functools

import jax
import jax.numpy as jnp
from jax import lax
from jax.experimental import pallas as pl
from jax.experimental.pallas import tpu as pltpu
from jax.experimental.pallas import tpu_sc as plsc


def _make_gather(B, D, chunk):
    info = plsc.get_sparse_core_info()
    NC, NS = info.num_cores, info.num_subcores
    NW = NC * NS
    b_per_w = B // NW
    n_chunks = b_per_w // chunk
    mesh = plsc.VectorSubcoreMesh(core_axis_name="c", subcore_axis_name="s")

    @functools.partial(
        pl.kernel,
        mesh=mesh,
        out_type=jax.ShapeDtypeStruct((B, D), jnp.float32),
        scratch_types=[
            pltpu.VMEM((chunk,), jnp.int32),
            pltpu.VMEM((chunk, D), jnp.float32),
            pltpu.SemaphoreType.DMA,
        ],
        compiler_params=pltpu.CompilerParams(use_tc_tiling_on_sc=False),
    )
    def gather_kernel(table_hbm, idx_hbm, out_hbm, idx_v, rows_v, sem):
        wid = lax.axis_index("s") * NC + lax.axis_index("c")
        base = wid * b_per_w
        for c in range(n_chunks):
            off = base + c * chunk
            pltpu.sync_copy(idx_hbm.at[pl.ds(off, chunk)], idx_v)
            pltpu.async_copy(table_hbm.at[idx_v], rows_v, sem).wait()
            pltpu.sync_copy(rows_v, out_hbm.at[pl.ds(off, chunk)])

    return gather_kernel


def kernel(input, table):
    seq, batch, _ = input.shape
    vocab, dim = table.shape
    B = seq * batch
    idx = input.reshape(B)
    out = _make_gather(B, dim, 800)(table, idx)
    return out.reshape(seq, batch, dim)


# speedup vs baseline: 1.0073x; 1.0073x over previous
import functools

import jax
import jax.numpy as jnp
from jax import lax
from jax.experimental import pallas as pl
from jax.experimental.pallas import tpu as pltpu
from jax.experimental.pallas import tpu_sc as plsc


def _make_gather(B, D, chunk):
    info = plsc.get_sparse_core_info()
    NC, NS = info.num_cores, info.num_subcores
    NW = NC * NS
    b_per_w = B // NW
    n_chunks = b_per_w // chunk
    mesh = plsc.VectorSubcoreMesh(core_axis_name="c", subcore_axis_name="s")

    @functools.partial(
        pl.kernel,
        mesh=mesh,
        out_type=jax.ShapeDtypeStruct((B, D), jnp.float32),
        scratch_types=[
            pltpu.VMEM((n_chunks, chunk), jnp.int32),
            pltpu.VMEM((chunk, D), jnp.float32),
            pltpu.VMEM((chunk, D), jnp.float32),
            pltpu.SemaphoreType.DMA,
            pltpu.SemaphoreType.DMA,
        ],
        compiler_params=pltpu.CompilerParams(use_tc_tiling_on_sc=False),
    )
    def gather_kernel(table_hbm, idx_hbm, out_hbm, idx_v, rows0, rows1, sem0, sem1):
        wid = lax.axis_index("s") * NC + lax.axis_index("c")
        base = wid * b_per_w
        bufs = (rows0, rows1)
        sems = (sem0, sem1)
        pltpu.sync_copy(idx_hbm.at[pl.ds(wid * n_chunks, n_chunks)], idx_v)
        prev = pltpu.async_copy(table_hbm.at[idx_v.at[0]], bufs[0], sems[0])
        for c in range(1, n_chunks):
            cur = pltpu.async_copy(
                table_hbm.at[idx_v.at[c]], bufs[c % 2], sems[c % 2]
            )
            prev.wait()
            pltpu.sync_copy(
                bufs[(c - 1) % 2], out_hbm.at[pl.ds(base + (c - 1) * chunk, chunk)]
            )
            prev = cur
        prev.wait()
        last = n_chunks - 1
        pltpu.sync_copy(
            bufs[last % 2], out_hbm.at[pl.ds(base + last * chunk, chunk)]
        )

    return gather_kernel


def kernel(input, table):
    seq, batch, _ = input.shape
    vocab, dim = table.shape
    B = seq * batch
    chunk = 800
    idx2d = input.reshape(B // chunk, chunk)
    out = _make_gather(B, dim, chunk)(table, idx2d)
    return out.reshape(seq, batch, dim)


# speedup vs baseline: 1.0088x; 1.0014x over previous
import functools

import jax
import jax.numpy as jnp
from jax import lax
from jax.experimental import pallas as pl
from jax.experimental.pallas import tpu as pltpu
from jax.experimental.pallas import tpu_sc as plsc


def _make_gather(B, D, chunk, nbuf):
    info = plsc.get_sparse_core_info()
    NC, NS = info.num_cores, info.num_subcores
    NW = NC * NS
    b_per_w = B // NW
    n_chunks = b_per_w // chunk
    mesh = plsc.VectorSubcoreMesh(core_axis_name="c", subcore_axis_name="s")

    scratch = [pltpu.VMEM((n_chunks, chunk), jnp.int32)]
    scratch += [pltpu.VMEM((chunk, D), jnp.float32) for _ in range(nbuf)]
    scratch += [pltpu.SemaphoreType.DMA for _ in range(2 * nbuf)]

    @functools.partial(
        pl.kernel,
        mesh=mesh,
        out_type=jax.ShapeDtypeStruct((B, D), jnp.float32),
        scratch_types=scratch,
        compiler_params=pltpu.CompilerParams(use_tc_tiling_on_sc=False),
    )
    def gather_kernel(table_hbm, idx_hbm, out_hbm, idx_v, *rest):
        bufs = rest[:nbuf]
        gsems = rest[nbuf : 2 * nbuf]
        wsems = rest[2 * nbuf : 3 * nbuf]
        wid = lax.axis_index("s") * NC + lax.axis_index("c")
        base = wid * b_per_w
        pltpu.sync_copy(idx_hbm.at[pl.ds(wid * n_chunks, n_chunks)], idx_v)
        g = [None] * nbuf
        w = [None] * nbuf

        def drain(d):
            s = d % nbuf
            g[s].wait()
            w[s] = pltpu.async_copy(
                bufs[s], out_hbm.at[pl.ds(base + d * chunk, chunk)], wsems[s]
            )

        for c in range(n_chunks):
            s = c % nbuf
            if c >= nbuf:
                w[s].wait()
            g[s] = pltpu.async_copy(table_hbm.at[idx_v.at[c]], bufs[s], gsems[s])
            if c >= nbuf - 1:
                drain(c - (nbuf - 1))
        for d in range(max(0, n_chunks - nbuf + 1), n_chunks):
            drain(d)
        for s in range(nbuf):
            if w[s] is not None:
                w[s].wait()

    return gather_kernel


def kernel(input, table):
    seq, batch, _ = input.shape
    vocab, dim = table.shape
    B = seq * batch
    chunk, nbuf = 200, 8
    idx2d = input.reshape(B // chunk, chunk)
    out = _make_gather(B, dim, chunk, nbuf)(table, idx2d)
    return out.reshape(seq, batch, dim)
